# baseline (speedup 1.0000x reference)
import functools

import jax
import jax.numpy as jnp
from jax import lax
from jax.experimental import pallas as pl
from jax.experimental.pallas import tpu as pltpu
from jax.experimental.pallas import tpu_sc as plsc

_NODE_NF = 1000000
_HIDDEN = 32
_BATCH = 16384
_FIELDS = 26

_NC = 2
_NS = 16
_NW = _NC * _NS
_ZROWS_PER_W = _BATCH // _NW
_K = 16
_GROUPS = _ZROWS_PER_W // _K

_mesh = plsc.VectorSubcoreMesh(core_axis_name="c", subcore_axis_name="s")


@functools.partial(
    pl.kernel,
    mesh=_mesh,
    compiler_params=pltpu.CompilerParams(use_tc_tiling_on_sc=False),
    out_type=jax.ShapeDtypeStruct((_BATCH, _FIELDS, _HIDDEN), jnp.float32),
    scratch_types=[
        pltpu.VMEM((_ZROWS_PER_W, _FIELDS), jnp.int32),
        pltpu.VMEM((_K, _FIELDS, _HIDDEN), jnp.float32),
        pltpu.VMEM((_K, _FIELDS, _HIDDEN), jnp.float32),
        pltpu.SemaphoreType.DMA,
        pltpu.SemaphoreType.DMA,
        pltpu.SemaphoreType.DMA,
        pltpu.SemaphoreType.DMA,
    ],
)
def _sc_gather(z_hbm, table_hbm, out_hbm, idx_v, bufa, bufb, gsa, gsb, osa, osb):
    wid = lax.axis_index("s") * _NC + lax.axis_index("c")
    zrow_base = wid * _ZROWS_PER_W

    pltpu.sync_copy(z_hbm.at[pl.ds(zrow_base, _ZROWS_PER_W)], idx_v)

    def fire_gathers(g, buf, sem):
        for j in range(_K):
            r = g * _K + j
            pltpu.async_copy(
                table_hbm.at[idx_v.at[r]], buf.at[j], sem
            )

    def drain_gathers(sem, buf):
        for j in range(_K):
            pltpu.make_async_copy(
                table_hbm.at[idx_v.at[0]], buf.at[j], sem
            ).wait()

    def fire_write(g, buf, sem):
        r0 = zrow_base + g * _K
        pltpu.async_copy(buf, out_hbm.at[pl.ds(r0, _K)], sem)

    def drain_write(g, buf, sem):
        r0 = zrow_base + g * _K
        pltpu.make_async_copy(buf, out_hbm.at[pl.ds(r0, _K)], sem).wait()

    fire_gathers(0, bufa, gsa)
    fire_gathers(1, bufb, gsb)

    def body(i, carry):
        ga = 2 * i
        drain_gathers(gsa, bufa)
        fire_write(ga, bufa, osa)
        drain_gathers(gsb, bufb)
        fire_write(ga + 1, bufb, osb)
        drain_write(ga, bufa, osa)
        fire_gathers(ga + 2, bufa, gsa)
        drain_write(ga + 1, bufb, osb)
        fire_gathers(ga + 3, bufb, gsb)
        return carry

    lax.fori_loop(0, _GROUPS // 2 - 1, body, 0)

    last = _GROUPS - 2
    drain_gathers(gsa, bufa)
    fire_write(last, bufa, osa)
    drain_gathers(gsb, bufb)
    fire_write(last + 1, bufb, osb)
    drain_write(last, bufa, osa)
    drain_write(last + 1, bufb, osb)


def kernel(z, table):
    return (_sc_gather(z, table), 0)
